# baseline (speedup 1.0000x reference)
import functools

import jax
import jax.numpy as jnp
from jax.experimental import pallas as pl
from jax.experimental.pallas import tpu as pltpu

N = 10000
E = 160000
R = 8
NB = 12
D = 256

BN = 400
NSTEP = N // BN


def _mm_body(cnt_ref, a_ref, x_ref, basis_ref, comp_ref, root_ref, bias_ref,
             o_ref, w_ref):
    i = pl.program_id(0)

    @pl.when(i == 0)
    def _():
        bflat = basis_ref[...].reshape(NB, D * D)
        wflat = jax.lax.dot(comp_ref[...], bflat)
        w_ref[0:R * D, :] = wflat.reshape(R * D, D)
        w_ref[R * D:R * D + D, :] = root_ref[...]

    cnt = cnt_ref[...]
    norm = 1.0 / jnp.maximum(cnt[0] + cnt[1], 1.0)
    a = a_ref[...] * norm[:, :, None]
    full = jnp.concatenate([a.reshape(BN, R * D), x_ref[...]], axis=1)
    o_ref[...] = jnp.maximum(jax.lax.dot(full, w_ref[...]) + bias_ref[...],
                             0.0)


@functools.partial(jax.jit, static_argnames=())
def _rgcn_layer_mm(cnt2, a3, x, basis, comp, root, bias2):
    return pl.pallas_call(
        _mm_body,
        grid=(NSTEP,),
        in_specs=[
            pl.BlockSpec((2, BN, R), lambda i: (0, i, 0)),
            pl.BlockSpec((BN, R, D), lambda i: (i, 0, 0)),
            pl.BlockSpec((BN, D), lambda i: (i, 0)),
            pl.BlockSpec((NB, D, D), lambda i: (0, 0, 0)),
            pl.BlockSpec((R, NB), lambda i: (0, 0)),
            pl.BlockSpec((D, D), lambda i: (0, 0)),
            pl.BlockSpec((1, D), lambda i: (0, 0)),
        ],
        out_specs=pl.BlockSpec((BN, D), lambda i: (i, 0)),
        out_shape=jax.ShapeDtypeStruct((N, D), jnp.float32),
        scratch_shapes=[pltpu.VMEM((R * D + D, D), jnp.float32)],
    )(cnt2, a3, x, basis, comp, root, bias2)


def kernel(adj_t, edge_types, emb, basis1, comp1, root1, bias1,
           basis2, comp2, root2, bias2, basis3, comp3, root3, bias3):
    src = adj_t[0]
    dst = adj_t[1]
    key = dst * R + edge_types

    cnt = jnp.zeros((N * R,), jnp.float32).at[key].add(1.0)
    cnt2 = jnp.stack([cnt, jnp.zeros_like(cnt)]).reshape(2, N, R)

    def seg_accum(x):
        a = jnp.zeros((N * R, D), jnp.float32).at[key].add(x[src])
        return a.reshape(N, R, D)

    x = emb
    outs = []
    for basis, comp, root, bias in ((basis1, comp1, root1, bias1),
                                    (basis2, comp2, root2, bias2),
                                    (basis3, comp3, root3, bias3)):
        a3 = seg_accum(x)
        x = _rgcn_layer_mm(cnt2, a3, x, basis, comp, root,
                           bias.reshape(1, D))
        outs.append(x)

    x1, x2, x3 = outs
    return jnp.concatenate((x3, x2, x1, emb), axis=1)
